# baseline (speedup 1.0000x reference)
import functools

import jax
import jax.numpy as jnp
from jax.experimental import pallas as pl
from jax.experimental.pallas import tpu as pltpu

_Q = 1024
_D = 16
_K = 100000
_KB = 4000
_NBLK = _K // _KB


def _nn_kernel(featt_ref, mb_ref, stats_ref, out_ref, nfeatt_ref, macc_ref):
    k = pl.program_id(0)

    @pl.when(k == 0)
    def _init():
        nfeatt_ref[...] = -2.0 * featt_ref[...]

    mbb = mb_ref[...]
    b_sq = jnp.sum(mbb * mbb, axis=1, keepdims=True)

    dots = jax.lax.dot_general(
        mbb, nfeatt_ref[...],
        dimension_numbers=(((1,), (0,)), ((), ())),
        preferred_element_type=jnp.float32,
    )
    sq = dots + b_sq
    m8 = jnp.min(sq.reshape(_KB // 8, 8, _Q), axis=0)

    @pl.when(k == 0)
    def _first():
        macc_ref[...] = m8

    @pl.when(k > 0)
    def _acc():
        macc_ref[...] = jnp.minimum(macc_ref[...], m8)

    @pl.when(k == _NBLK - 1)
    def _finish():
        featt = featt_ref[...]
        a_sq = jnp.sum(featt * featt, axis=0, keepdims=True)
        row_min = jnp.min(macc_ref[...], axis=0, keepdims=True)
        sq_min = jnp.maximum(row_min + a_sq, 1e-12)
        dist = jnp.sqrt(sq_min)
        s_min = stats_ref[0]
        s_max = stats_ref[1]
        val = (dist - s_min) / (s_max - s_min)
        out_ref[...] = jnp.broadcast_to(val, (8, _Q))


@functools.partial(jax.jit, static_argnames=())
def _run(features, memory_bank, stats):
    featt = features.T
    out = pl.pallas_call(
        _nn_kernel,
        grid=(_NBLK,),
        in_specs=[
            pl.BlockSpec((_D, _Q), lambda k: (0, 0)),
            pl.BlockSpec((_KB, _D), lambda k: (k, 0)),
            pl.BlockSpec(memory_space=pltpu.SMEM),
        ],
        out_specs=pl.BlockSpec((8, _Q), lambda k: (0, 0)),
        out_shape=jax.ShapeDtypeStruct((8, _Q), jnp.float32),
        scratch_shapes=[
            pltpu.VMEM((_D, _Q), jnp.float32),
            pltpu.VMEM((8, _Q), jnp.float32),
        ],
    )(featt, memory_bank, stats)
    return out[0]


def kernel(features, memory_bank, stats_min, stats_max):
    stats = jnp.stack([jnp.asarray(stats_min, jnp.float32),
                       jnp.asarray(stats_max, jnp.float32)])
    return _run(features, memory_bank, stats)


# speedup vs baseline: 1.2252x; 1.2252x over previous
import functools

import jax
import jax.numpy as jnp
from jax.experimental import pallas as pl
from jax.experimental.pallas import tpu as pltpu

_Q = 1024
_D = 16
_DA = 32
_K = 100000
_KB = 4096
_NBLK = 26
_K_PAD = _KB * _NBLK


def _nn_kernel(featt_ref, mbt_ref, stats_ref, out_ref, w_ref, aug_ref,
               macc_ref):
    k = pl.program_id(0)

    @pl.when(k == 0)
    def _init():
        featt = featt_ref[...].astype(jnp.float32)
        a_sq = jnp.sum(featt * featt, axis=0, keepdims=True)
        a_hi = a_sq.astype(jnp.bfloat16)
        a_lo = (a_sq - a_hi.astype(jnp.float32)).astype(jnp.bfloat16)
        w_ref[...] = jnp.zeros((_DA, _Q), jnp.bfloat16)
        w_ref[0:_D, :] = (-2.0 * featt).astype(jnp.bfloat16)
        w_ref[_D:_D + 2, :] = jnp.ones((2, _Q), jnp.bfloat16)
        w_ref[_D + 2:_D + 3, :] = a_hi
        w_ref[_D + 3:_D + 4, :] = a_lo
        aug_ref[...] = jnp.zeros((2, _DA, _KB), jnp.bfloat16)

    mbt = mbt_ref[...]
    mbtf = mbt.astype(jnp.float32)
    b_sq = jnp.sum(mbtf * mbtf, axis=0, keepdims=True)
    b_hi = b_sq.astype(jnp.bfloat16)
    b_lo = (b_sq - b_hi.astype(jnp.float32)).astype(jnp.bfloat16)
    col = jax.lax.broadcasted_iota(jnp.int32, (1, _KB), 1) + k * _KB
    valid = col < _K
    b_hi = jnp.where(valid, b_hi, jnp.inf).astype(jnp.bfloat16)
    b_lo = jnp.where(valid, b_lo, 0.0).astype(jnp.bfloat16)

    extra = jnp.concatenate(
        [b_hi, b_lo,
         jnp.ones((2, _KB), jnp.bfloat16),
         jnp.zeros((4, _KB), jnp.bfloat16)], axis=0)
    bi = k % 2
    aug_ref[bi, 0:_D, :] = mbt
    aug_ref[bi, _D:_D + 8, :] = extra

    @pl.when(k > 0)
    def _dot():
        aug = aug_ref[(k - 1) % 2]
        sq = jax.lax.dot_general(
            aug, w_ref[...],
            dimension_numbers=(((0,), (0,)), ((), ())),
            preferred_element_type=jnp.float32,
        )
        m8 = jnp.min(sq.reshape(_KB // 8, 8, _Q), axis=0)

        @pl.when(k == 1)
        def _first():
            macc_ref[...] = m8

        @pl.when(k > 1)
        def _acc():
            macc_ref[...] = jnp.minimum(macc_ref[...], m8)

    @pl.when(k == _NBLK - 1)
    def _finish():
        row_min = jnp.min(macc_ref[...], axis=0, keepdims=True)
        sq_min = jnp.maximum(row_min, 1e-12)
        dist = jnp.sqrt(sq_min)
        s_min = stats_ref[0]
        s_max = stats_ref[1]
        val = (dist - s_min) / (s_max - s_min)
        out_ref[...] = jnp.broadcast_to(val, (8, _Q))


@functools.partial(jax.jit, static_argnames=())
def _run(features, memory_bank, stats):
    featt = features.T.astype(jnp.bfloat16)
    mbt = jnp.pad(memory_bank,
                  ((0, _K_PAD - _K), (0, 0))).T.astype(jnp.bfloat16)
    out = pl.pallas_call(
        _nn_kernel,
        grid=(_NBLK,),
        in_specs=[
            pl.BlockSpec((_D, _Q), lambda k: (0, 0)),
            pl.BlockSpec((_D, _KB), lambda k: (0, k)),
            pl.BlockSpec(memory_space=pltpu.SMEM),
        ],
        out_specs=pl.BlockSpec((8, _Q), lambda k: (0, 0)),
        out_shape=jax.ShapeDtypeStruct((8, _Q), jnp.float32),
        scratch_shapes=[
            pltpu.VMEM((_DA, _Q), jnp.bfloat16),
            pltpu.VMEM((2, _DA, _KB), jnp.bfloat16),
            pltpu.VMEM((8, _Q), jnp.float32),
        ],
    )(featt, mbt, stats)
    return out[0]


def kernel(features, memory_bank, stats_min, stats_max):
    stats = jnp.stack([jnp.asarray(stats_min, jnp.float32),
                       jnp.asarray(stats_max, jnp.float32)])
    return _run(features, memory_bank, stats)


# speedup vs baseline: 1.2569x; 1.0258x over previous
import functools

import jax
import jax.numpy as jnp
from jax.experimental import pallas as pl
from jax.experimental.pallas import tpu as pltpu

_Q = 1024
_D = 16
_K = 100000
_KB = 8192
_K_PAD = 106496
_NBLK = _K_PAD // _KB


def _nn_kernel(featt_ref, mbt_ref, stats_ref, out_ref, nfeatt_ref, macc_ref):
    k = pl.program_id(0)

    @pl.when(k == 0)
    def _init():
        nfeatt_ref[...] = (-2.0 * featt_ref[...].astype(jnp.float32)
                           ).astype(jnp.bfloat16)

    mbt = mbt_ref[...]
    mbtf = mbt.astype(jnp.float32)
    b_sq_row = jnp.sum(mbtf * mbtf, axis=0, keepdims=True)
    col = jax.lax.broadcasted_iota(jnp.int32, (1, _KB), 1) + k * _KB
    b_sq_row = jnp.where(col < _K, b_sq_row, jnp.inf)
    b_sq = b_sq_row.reshape(_KB, 1)

    dots = jax.lax.dot_general(
        mbt, nfeatt_ref[...],
        dimension_numbers=(((0,), (0,)), ((), ())),
        preferred_element_type=jnp.float32,
    )
    sq = dots + b_sq
    m8 = jnp.min(sq.reshape(_KB // 8, 8, _Q), axis=0)

    @pl.when(k == 0)
    def _first():
        macc_ref[...] = m8

    @pl.when(k > 0)
    def _acc():
        macc_ref[...] = jnp.minimum(macc_ref[...], m8)

    @pl.when(k == _NBLK - 1)
    def _finish():
        featt = featt_ref[...].astype(jnp.float32)
        a_sq = jnp.sum(featt * featt, axis=0, keepdims=True)
        row_min = jnp.min(macc_ref[...], axis=0, keepdims=True)
        sq_min = jnp.maximum(row_min + a_sq, 1e-12)
        dist = jnp.sqrt(sq_min)
        s_min = stats_ref[0]
        s_max = stats_ref[1]
        val = (dist - s_min) / (s_max - s_min)
        out_ref[...] = jnp.broadcast_to(val, (8, _Q))


@functools.partial(jax.jit, static_argnames=())
def _run(features, memory_bank, stats):
    featt = features.T.astype(jnp.bfloat16)
    mbt = jnp.pad(memory_bank, ((0, _K_PAD - _K), (0, 0))).T.astype(jnp.bfloat16)
    out = pl.pallas_call(
        _nn_kernel,
        grid=(_NBLK,),
        in_specs=[
            pl.BlockSpec((_D, _Q), lambda k: (0, 0)),
            pl.BlockSpec((_D, _KB), lambda k: (0, k)),
            pl.BlockSpec(memory_space=pltpu.SMEM),
        ],
        out_specs=pl.BlockSpec((8, _Q), lambda k: (0, 0)),
        out_shape=jax.ShapeDtypeStruct((8, _Q), jnp.float32),
        scratch_shapes=[
            pltpu.VMEM((_D, _Q), jnp.bfloat16),
            pltpu.VMEM((8, _Q), jnp.float32),
        ],
    )(featt, mbt, stats)
    return out[0]


def kernel(features, memory_bank, stats_min, stats_max):
    stats = jnp.stack([jnp.asarray(stats_min, jnp.float32),
                       jnp.asarray(stats_max, jnp.float32)])
    return _run(features, memory_bank, stats)
